# baseline (speedup 1.0000x reference)
import functools

import jax
import jax.numpy as jnp
from jax import lax
from jax.experimental import pallas as pl
from jax.experimental.pallas import tpu as pltpu
from jax.experimental.pallas import tpu_sc as plsc

_NC = 2
_NS = 16
_NW = _NC * _NS

_DIM = 128
_TOTAL = 4096 * 50
_PER_W = _TOTAL // _NW
_CHUNK = 256
_NCHUNK = _PER_W // _CHUNK
_NBUF = 3
_NWIF = 1


def _embed_body(idx_hbm, table_hbm, out_hbm, idx_v, *scratch):
    rows = scratch[:_NBUF]
    gsems = scratch[_NBUF:2 * _NBUF]
    wsems = scratch[2 * _NBUF:3 * _NBUF]
    wid = lax.axis_index("s") * _NC + lax.axis_index("c")
    base = wid * _PER_W

    pltpu.sync_copy(idx_hbm.at[pl.ds(base, _PER_W)], idx_v)

    def fire_gather(k, b):
        pltpu.async_copy(table_hbm.at[idx_v.at[pl.ds(k * _CHUNK, _CHUNK)]],
                         rows[b], gsems[b])

    def drain_gather(b):
        pltpu.make_async_copy(table_hbm.at[pl.ds(0, _CHUNK)], rows[b],
                              gsems[b]).wait()

    def fire_write(k, b):
        pltpu.async_copy(
            rows[b], out_hbm.at[pl.ds(base + k * _CHUNK, _CHUNK)], wsems[b])

    def drain_write(k, b):
        pltpu.make_async_copy(
            rows[b], out_hbm.at[pl.ds(base + k * _CHUNK, _CHUNK)],
            wsems[b]).wait()

    def visit(j, b, b_free):
        if b_free is not None:
            drain_write(j - _NWIF, b_free)
            fire_gather(j - _NWIF + _NBUF, b_free)
        drain_gather(b)
        fire_write(j, b)

    for b in range(_NBUF):
        fire_gather(b, b)

    for j in range(_NWIF):
        visit(j, j % _NBUF, None)

    n_full = _NCHUNK - _NBUF
    n_groups = n_full // _NBUF
    if n_groups:
        @pl.loop(_NWIF, _NWIF + n_groups * _NBUF, step=_NBUF)
        def _(c):
            for u in range(_NBUF):
                visit(c + u, (_NWIF + u) % _NBUF, u % _NBUF)

    for j in range(_NWIF + n_groups * _NBUF, _NWIF + n_full):
        visit(j, j % _NBUF, (j - _NWIF) % _NBUF)

    for j in range(_NWIF + n_full, _NCHUNK):
        drain_write(j - _NWIF, (j - _NWIF) % _NBUF)
        visit(j, j % _NBUF, None)

    for k in range(_NCHUNK - _NWIF, _NCHUNK):
        drain_write(k, k % _NBUF)


@jax.jit
def _embed(idx, table):
    mesh = plsc.VectorSubcoreMesh(core_axis_name="c", subcore_axis_name="s")
    f = pl.kernel(
        _embed_body,
        out_type=jax.ShapeDtypeStruct((_TOTAL, _DIM), jnp.float32),
        mesh=mesh,
        compiler_params=pltpu.CompilerParams(use_tc_tiling_on_sc=False),
        scratch_types=(
            [pltpu.VMEM((_PER_W,), jnp.int32)]
            + [pltpu.VMEM((_CHUNK, _DIM), jnp.float32)] * _NBUF
            + [pltpu.SemaphoreType.DMA] * (2 * _NBUF)
        ),
    )
    return f(idx, table)


def kernel(index, table):
    b, l = index.shape
    idx = index.astype(jnp.int32).reshape(_TOTAL)
    out = _embed(idx, table)
    return out.reshape(b, l, table.shape[1])


# speedup vs baseline: 3.1185x; 3.1185x over previous
import functools

import jax
import jax.numpy as jnp
from jax import lax
from jax.experimental import pallas as pl
from jax.experimental.pallas import tpu as pltpu
from jax.experimental.pallas import tpu_sc as plsc

_NC = 2
_NS = 16
_NW = _NC * _NS

_DIM = 128
_TOTAL = 4096 * 50
_PER_W = _TOTAL // _NW
_CHUNK = 256
_NCHUNK = _PER_W // _CHUNK
_NBUF = 3
_NWIF = 1


def _embed_body(idx_hbm, table_hbm, out_hbm, idx_v, *scratch):
    rows = scratch[:_NBUF]
    gsems = scratch[_NBUF:2 * _NBUF]
    wsems = scratch[2 * _NBUF:3 * _NBUF]
    wid = lax.axis_index("s") * _NC + lax.axis_index("c")
    base = wid * _PER_W

    pltpu.sync_copy(idx_hbm.at[pl.ds(base, _PER_W)], idx_v)

    def fire_gather(k, b):
        pltpu.async_copy(table_hbm.at[idx_v.at[pl.ds(k * _CHUNK, _CHUNK)]],
                         rows[b], gsems[b])

    def drain_gather(b):
        pltpu.make_async_copy(table_hbm.at[pl.ds(0, _CHUNK)], rows[b],
                              gsems[b]).wait()

    def fire_write(k, b):
        pltpu.async_copy(
            rows[b], out_hbm.at[pl.ds(base + k * _CHUNK, _CHUNK)], wsems[b])

    def drain_write(k, b):
        pltpu.make_async_copy(
            rows[b], out_hbm.at[pl.ds(base + k * _CHUNK, _CHUNK)],
            wsems[b]).wait()

    def visit(j, b, b_free):
        if b_free is not None:
            drain_write(j - _NWIF, b_free)
            fire_gather(j - _NWIF + _NBUF, b_free)
        drain_gather(b)
        fire_write(j, b)

    for b in range(_NBUF):
        fire_gather(b, b)

    for j in range(_NWIF):
        visit(j, j % _NBUF, None)

    n_full = _NCHUNK - _NBUF
    n_groups = n_full // _NBUF
    if n_groups:
        @pl.loop(_NWIF, _NWIF + n_groups * _NBUF, step=_NBUF)
        def _(c):
            for u in range(_NBUF):
                visit(c + u, (_NWIF + u) % _NBUF, u % _NBUF)

    for j in range(_NWIF + n_groups * _NBUF, _NWIF + n_full):
        visit(j, j % _NBUF, (j - _NWIF) % _NBUF)

    for j in range(_NWIF + n_full, _NCHUNK):
        drain_write(j - _NWIF, (j - _NWIF) % _NBUF)
        visit(j, j % _NBUF, None)

    for k in range(_NCHUNK - _NWIF, _NCHUNK):
        drain_write(k, k % _NBUF)


@jax.jit
def _embed(idx, table):
    mesh = plsc.VectorSubcoreMesh(core_axis_name="c", subcore_axis_name="s")
    f = pl.kernel(
        _embed_body,
        out_type=jax.ShapeDtypeStruct((_TOTAL, _DIM), jnp.float32),
        mesh=mesh,
        compiler_params=pltpu.CompilerParams(use_tc_tiling_on_sc=False),
        scratch_types=(
            [pltpu.VMEM((_PER_W,), jnp.int32)]
            + [pltpu.VMEM((_CHUNK, _DIM), jnp.float32)] * _NBUF
            + [pltpu.SemaphoreType.DMA] * (2 * _NBUF)
        ),
    )
    return f(idx, table)


def kernel(index, table):
    b, l = index.shape
    idx = jnp.transpose(index.astype(jnp.int32)).reshape(_TOTAL)
    out = _embed(idx, table)
    return jnp.transpose(out.reshape(l, b, table.shape[1]), (1, 0, 2))


# speedup vs baseline: 3.1443x; 1.0083x over previous
import functools

import jax
import jax.numpy as jnp
from jax import lax
from jax.experimental import pallas as pl
from jax.experimental.pallas import tpu as pltpu
from jax.experimental.pallas import tpu_sc as plsc

_NC = 2
_NS = 16
_NW = _NC * _NS

_DIM = 128
_TOTAL = 4096 * 50
_PER_W = _TOTAL // _NW
_CHUNK = 128
_NCHUNK = _PER_W // _CHUNK
_NBUF = 7
_NWIF = 3


def _embed_body(idx_hbm, table_hbm, out_hbm, idx_v, *scratch):
    rows = scratch[:_NBUF]
    gsems = scratch[_NBUF:2 * _NBUF]
    wsems = scratch[2 * _NBUF:3 * _NBUF]
    wid = lax.axis_index("s") * _NC + lax.axis_index("c")
    base = wid * _PER_W

    pltpu.sync_copy(idx_hbm.at[pl.ds(base, _PER_W)], idx_v)

    def fire_gather(k, b):
        pltpu.async_copy(table_hbm.at[idx_v.at[pl.ds(k * _CHUNK, _CHUNK)]],
                         rows[b], gsems[b])

    def drain_gather(b):
        pltpu.make_async_copy(table_hbm.at[pl.ds(0, _CHUNK)], rows[b],
                              gsems[b]).wait()

    def fire_write(k, b):
        pltpu.async_copy(
            rows[b], out_hbm.at[pl.ds(base + k * _CHUNK, _CHUNK)], wsems[b])

    def drain_write(k, b):
        pltpu.make_async_copy(
            rows[b], out_hbm.at[pl.ds(base + k * _CHUNK, _CHUNK)],
            wsems[b]).wait()

    def visit(j, b, b_free):
        if b_free is not None:
            drain_write(j - _NWIF, b_free)
            fire_gather(j - _NWIF + _NBUF, b_free)
        drain_gather(b)
        fire_write(j, b)

    for b in range(_NBUF):
        fire_gather(b, b)

    for j in range(_NWIF):
        visit(j, j % _NBUF, None)

    n_full = _NCHUNK - _NBUF
    n_groups = n_full // _NBUF
    if n_groups:
        @pl.loop(_NWIF, _NWIF + n_groups * _NBUF, step=_NBUF)
        def _(c):
            for u in range(_NBUF):
                visit(c + u, (_NWIF + u) % _NBUF, u % _NBUF)

    for j in range(_NWIF + n_groups * _NBUF, _NWIF + n_full):
        visit(j, j % _NBUF, (j - _NWIF) % _NBUF)

    for j in range(_NWIF + n_full, _NCHUNK):
        drain_write(j - _NWIF, (j - _NWIF) % _NBUF)
        visit(j, j % _NBUF, None)

    for k in range(_NCHUNK - _NWIF, _NCHUNK):
        drain_write(k, k % _NBUF)


@jax.jit
def _embed(idx, table):
    mesh = plsc.VectorSubcoreMesh(core_axis_name="c", subcore_axis_name="s")
    f = pl.kernel(
        _embed_body,
        out_type=jax.ShapeDtypeStruct((_TOTAL, _DIM), jnp.float32),
        mesh=mesh,
        compiler_params=pltpu.CompilerParams(use_tc_tiling_on_sc=False),
        scratch_types=(
            [pltpu.VMEM((_PER_W,), jnp.int32)]
            + [pltpu.VMEM((_CHUNK, _DIM), jnp.float32)] * _NBUF
            + [pltpu.SemaphoreType.DMA] * (2 * _NBUF)
        ),
    )
    return f(idx, table)


def kernel(index, table):
    b, l = index.shape
    idx = jnp.transpose(index.astype(jnp.int32)).reshape(_TOTAL)
    out = _embed(idx, table)
    return jnp.transpose(out.reshape(l, b, table.shape[1]), (1, 0, 2))
